# baseline (speedup 1.0000x reference)
import jax
import jax.numpy as jnp
from jax import lax
from jax.experimental import pallas as pl
from jax.experimental.pallas import tpu as pltpu
from jax.experimental.pallas import tpu_sc as plsc

N = 50000
E = 800000
D = 88
H = 128
SH = 4
NS = 32
DP = 96
NCH = 6
CW = 16
NCORE = 2
NSUB = 16

EB = 2000
NB = 2000

PER_CORE = E // NCORE
PER_TILE = PER_CORE // NSUB
NFULL = 195
TAIL = PER_TILE - NFULL * 128
NP8 = 50048
ACC_STRIPE = NP8 // NSUB


def _k1_body(ef_ref, esh_ref, w1_ref, b1_ref, w2_ref, b2_ref, wsh_ref,
             *m_refs):
    h = jnp.maximum(ef_ref[...] @ w1_ref[...] + b1_ref[...], 0.0)
    w = h @ w2_ref[...] + b2_ref[...]
    shp = esh_ref[...] @ wsh_ref[...]
    m = w * shp
    lane = lax.broadcasted_iota(jnp.int32, m.shape, 1)
    m = jnp.where(lane == D, 1.0, jnp.where(lane > D, 0.0, m))
    for c in range(NCH):
        m_refs[c][...] = m[:, c * CW:(c + 1) * CW]


def _k1(ef, esh, w1, b1r, w2p, b2p, wshp):
    return pl.pallas_call(
        _k1_body,
        grid=(E // EB,),
        in_specs=[
            pl.BlockSpec((EB, H), lambda i: (i, 0)),
            pl.BlockSpec((EB, SH), lambda i: (i, 0)),
            pl.BlockSpec((H, H), lambda i: (0, 0)),
            pl.BlockSpec((1, H), lambda i: (0, 0)),
            pl.BlockSpec((H, DP), lambda i: (0, 0)),
            pl.BlockSpec((1, DP), lambda i: (0, 0)),
            pl.BlockSpec((SH, DP), lambda i: (0, 0)),
        ],
        out_specs=[pl.BlockSpec((EB, CW), lambda i: (i, 0))] * NCH,
        out_shape=[jax.ShapeDtypeStruct((E, CW), jnp.float32)] * NCH,
    )(ef, esh, w1, b1r, w2p, b2p, wshp)


def _k3a_body(sa_ref, sb_ref, x_ref, out_ref, ssq_ref):
    i = pl.program_id(0)
    s = sa_ref[...] + sb_ref[...]
    cnt = jnp.maximum(s[:, D:D + 1], 1.0)
    out = s[:, :D] / cnt + x_ref[...]
    out_ref[...] = out

    @pl.when(i == 0)
    def _init():
        ssq_ref[...] = jnp.zeros_like(ssq_ref)

    ssq_ref[...] += jnp.sum(out * out, axis=0, keepdims=True)


def _k3a(sums_a, sums_b, x):
    return pl.pallas_call(
        _k3a_body,
        grid=(N // NB,),
        in_specs=[
            pl.BlockSpec((NB, DP), lambda i: (i, 0)),
            pl.BlockSpec((NB, DP), lambda i: (i, 0)),
            pl.BlockSpec((NB, D), lambda i: (i, 0)),
        ],
        out_specs=[
            pl.BlockSpec((NB, D), lambda i: (i, 0)),
            pl.BlockSpec((1, D), lambda i: (0, 0)),
        ],
        out_shape=[
            jax.ShapeDtypeStruct((N, D), jnp.float32),
            jax.ShapeDtypeStruct((1, D), jnp.float32),
        ],
    )(sums_a, sums_b, x)


def _k3b_body(ou_ref, ssq_ref, wlin_ref, out_ref, s_ref):
    scale = lax.rsqrt(ssq_ref[...] / N + 1e-5)
    out = ou_ref[...] * scale
    out_ref[...] = out
    s_ref[...] = out @ wlin_ref[...]


def _k3b(out_un, ssq, wlin):
    return pl.pallas_call(
        _k3b_body,
        grid=(N // NB,),
        in_specs=[
            pl.BlockSpec((NB, D), lambda i: (i, 0)),
            pl.BlockSpec((1, D), lambda i: (0, 0)),
            pl.BlockSpec((D, NS), lambda i: (0, 0)),
        ],
        out_specs=[
            pl.BlockSpec((NB, D), lambda i: (i, 0)),
            pl.BlockSpec((NB, NS), lambda i: (i, 0)),
        ],
        out_shape=[
            jax.ShapeDtypeStruct((N, D), jnp.float32),
            jax.ShapeDtypeStruct((N, NS), jnp.float32),
        ],
    )(out_un, ssq, wlin)


def _k5_body(sd_ref, ss_ref, ef_ref, wa_ref, wb_ref, wc_ref, bu1_ref,
             wu2_ref, bu2_ref, g_ref, b_ref, ef2_ref):
    ef = ef_ref[...]
    pre = (sd_ref[...] @ wa_ref[...] + ss_ref[...] @ wb_ref[...]
           + ef @ wc_ref[...] + bu1_ref[...])
    h1 = jnp.maximum(pre, 0.0)
    h = h1 @ wu2_ref[...] + bu2_ref[...]
    ef2 = ef + h
    mu = jnp.mean(ef2, axis=-1, keepdims=True)
    d = ef2 - mu
    var = jnp.mean(d * d, axis=-1, keepdims=True)
    ef2_ref[...] = g_ref[...] * d * lax.rsqrt(var + 1e-5) + b_ref[...]


def _k5(sd, ss, ef, wa, wb, wc, bu1r, wu2, bu2r, gr, br):
    return pl.pallas_call(
        _k5_body,
        grid=(E // EB,),
        in_specs=[
            pl.BlockSpec((EB, NS), lambda i: (i, 0)),
            pl.BlockSpec((EB, NS), lambda i: (i, 0)),
            pl.BlockSpec((EB, H), lambda i: (i, 0)),
            pl.BlockSpec((NS, H), lambda i: (0, 0)),
            pl.BlockSpec((NS, H), lambda i: (0, 0)),
            pl.BlockSpec((H, H), lambda i: (0, 0)),
            pl.BlockSpec((1, H), lambda i: (0, 0)),
            pl.BlockSpec((H, H), lambda i: (0, 0)),
            pl.BlockSpec((1, H), lambda i: (0, 0)),
            pl.BlockSpec((1, H), lambda i: (0, 0)),
            pl.BlockSpec((1, H), lambda i: (0, 0)),
        ],
        out_specs=pl.BlockSpec((EB, H), lambda i: (i, 0)),
        out_shape=jax.ShapeDtypeStruct((E, H), jnp.float32),
    )(sd, ss, ef, wa, wb, wc, bu1r, wu2, bu2r, gr, br)


def _sc_scatter_body(dst_h, src_h, x0, x1, x2, x3, x4, x5,
                     m0, m1, m2, m3, m4, m5, sums_h,
                     acc, dv, sv, dvt, svt, xr, mr, pr, zb, sem):
    core = lax.axis_index("c")
    tec = lax.axis_index("s")
    xs = (x0, x1, x2, x3, x4, x5)
    ms = (m0, m1, m2, m3, m4, m5)
    zero16 = jnp.zeros((CW,), jnp.float32)
    for r in range(128):
        zb[r] = zero16
    base = core * PER_CORE + tec * PER_TILE
    for p in range(NCH):
        x_h = xs[p]
        m_h = ms[p]
        for k in range(24):
            pltpu.sync_copy(zb, acc.at[pl.ds(tec * ACC_STRIPE + k * 128, 128)])
        pltpu.sync_copy(zb.at[pl.ds(0, 56)],
                        acc.at[pl.ds(tec * ACC_STRIPE + 24 * 128, 56)])
        plsc.subcore_barrier()

        def blk(i, carry):
            e0 = base + i * 128
            pltpu.sync_copy(dst_h.at[pl.ds(e0, 128)], dv)
            pltpu.sync_copy(src_h.at[pl.ds(e0, 128)], sv)
            pltpu.async_copy(x_h.at[dv], xr, sem).wait()
            pltpu.sync_copy(m_h.at[pl.ds(e0, 128)], mr)
            for r in range(128):
                pr[r] = xr[r] * mr[r]
            pltpu.sync_copy(pr, acc.at[sv], add=True)
            return carry

        lax.fori_loop(0, NFULL, blk, 0)
        e0 = base + NFULL * 128
        pltpu.sync_copy(dst_h.at[pl.ds(e0, TAIL)], dvt)
        pltpu.sync_copy(src_h.at[pl.ds(e0, TAIL)], svt)
        pltpu.async_copy(x_h.at[dvt], xr.at[pl.ds(0, TAIL)], sem).wait()
        pltpu.sync_copy(m_h.at[pl.ds(e0, TAIL)], mr.at[pl.ds(0, TAIL)])
        for r in range(TAIL):
            pr[r] = xr[r] * mr[r]
        pltpu.sync_copy(pr.at[pl.ds(0, TAIL)], acc.at[svt], add=True)
        plsc.subcore_barrier()
        pltpu.sync_copy(
            acc.at[pl.ds(tec * ACC_STRIPE, ACC_STRIPE)],
            sums_h.at[pl.ds((core * NCH + p) * NP8 + tec * ACC_STRIPE,
                            ACC_STRIPE)])
        plsc.subcore_barrier()


def _sc_scatter(dst, src, xs, ms):
    mesh = plsc.VectorSubcoreMesh(core_axis_name="c", subcore_axis_name="s")
    f = pl.kernel(
        _sc_scatter_body,
        out_type=jax.ShapeDtypeStruct((NCORE * NCH * NP8, CW), jnp.float32),
        mesh=mesh,
        scratch_types=[
            pltpu.VMEM_SHARED((NP8, CW), jnp.float32),
            pltpu.VMEM((128,), jnp.int32),
            pltpu.VMEM((128,), jnp.int32),
            pltpu.VMEM((TAIL,), jnp.int32),
            pltpu.VMEM((TAIL,), jnp.int32),
            pltpu.VMEM((128, CW), jnp.float32),
            pltpu.VMEM((128, CW), jnp.float32),
            pltpu.VMEM((128, CW), jnp.float32),
            pltpu.VMEM((128, CW), jnp.float32),
            pltpu.SemaphoreType.DMA,
        ],
        compiler_params=pltpu.CompilerParams(use_tc_tiling_on_sc=False),
    )
    return f(dst, src, *xs, *ms)


def _sc_gather_body(dst_h, src_h, s_h, sd_h, ss_h, iv, ivt, rows, sem):
    core = lax.axis_index("c")
    tec = lax.axis_index("s")
    base = core * PER_CORE + tec * PER_TILE

    def blk(i, carry):
        e0 = base + i * 128
        pltpu.sync_copy(dst_h.at[pl.ds(e0, 128)], iv)
        pltpu.async_copy(s_h.at[iv], rows, sem).wait()
        pltpu.sync_copy(rows, sd_h.at[pl.ds(e0, 128)])
        pltpu.sync_copy(src_h.at[pl.ds(e0, 128)], iv)
        pltpu.async_copy(s_h.at[iv], rows, sem).wait()
        pltpu.sync_copy(rows, ss_h.at[pl.ds(e0, 128)])
        return carry

    lax.fori_loop(0, NFULL, blk, 0)
    e0 = base + NFULL * 128
    pltpu.sync_copy(dst_h.at[pl.ds(e0, TAIL)], ivt)
    pltpu.async_copy(s_h.at[ivt], rows.at[pl.ds(0, TAIL)], sem).wait()
    pltpu.sync_copy(rows.at[pl.ds(0, TAIL)], sd_h.at[pl.ds(e0, TAIL)])
    pltpu.sync_copy(src_h.at[pl.ds(e0, TAIL)], ivt)
    pltpu.async_copy(s_h.at[ivt], rows.at[pl.ds(0, TAIL)], sem).wait()
    pltpu.sync_copy(rows.at[pl.ds(0, TAIL)], ss_h.at[pl.ds(e0, TAIL)])


def _sc_gather(dst, src, s):
    mesh = plsc.VectorSubcoreMesh(core_axis_name="c", subcore_axis_name="s")
    f = pl.kernel(
        _sc_gather_body,
        out_type=(
            jax.ShapeDtypeStruct((E, NS), jnp.float32),
            jax.ShapeDtypeStruct((E, NS), jnp.float32),
        ),
        mesh=mesh,
        scratch_types=[
            pltpu.VMEM((128,), jnp.int32),
            pltpu.VMEM((TAIL,), jnp.int32),
            pltpu.VMEM((128, NS), jnp.float32),
            pltpu.SemaphoreType.DMA,
        ],
        compiler_params=pltpu.CompilerParams(use_tc_tiling_on_sc=False),
    )
    return f(dst, src, s)


def _chunk_tables(x):
    xp = jnp.concatenate(
        [x, jnp.ones((N, 1), jnp.float32),
         jnp.zeros((N, DP - D - 1), jnp.float32)], axis=1)
    return [xp[:, c * CW:(c + 1) * CW] for c in range(NCH)]


def kernel(atom_features, edge_features, edge_sh, edge_index, W_sh, W1, b1,
           W2, b2, W_lin, Wu1, bu1, Wu2, bu2, gamma, beta):
    dst = edge_index[0]
    src = edge_index[1]

    w2p = jnp.pad(W2, ((0, 0), (0, DP - D)))
    b2p = jnp.pad(b2, (0, DP - D)).reshape(1, DP)
    wshp = jnp.pad(W_sh, ((0, 0), (0, DP - D)))
    b1r = b1.reshape(1, H)
    wa, wb, wc = Wu1[:NS], Wu1[NS:2 * NS], Wu1[2 * NS:]
    bu1r = bu1.reshape(1, H)
    bu2r = bu2.reshape(1, H)
    gr = gamma.reshape(1, H)
    br = beta.reshape(1, H)

    x = atom_features
    ef = edge_features
    for layer in range(2):
        ms = _k1(ef, edge_sh, W1, b1r, w2p, b2p, wshp)
        xs = _chunk_tables(x)
        sums_flat = _sc_scatter(dst, src, xs, ms)
        sums = sums_flat.reshape(NCORE, NCH, NP8, CW)[:, :, :N]
        sums_a = sums[0].transpose(1, 0, 2).reshape(N, DP)
        sums_b = sums[1].transpose(1, 0, 2).reshape(N, DP)
        out_un, ssq = _k3a(sums_a, sums_b, x)
        out, s = _k3b(out_un, ssq, W_lin)
        x = out
        if layer == 0:
            sd, ss = _sc_gather(dst, src, s)
            ef = _k5(sd, ss, ef, wa, wb, wc, bu1r, Wu2, bu2r, gr, br)
    return x


# speedup vs baseline: 2.4443x; 2.4443x over previous
import jax
import jax.numpy as jnp
from jax import lax
from jax.experimental import pallas as pl
from jax.experimental.pallas import tpu as pltpu
from jax.experimental.pallas import tpu_sc as plsc

N = 50000
E = 800000
D = 88
H = 128
SH = 4
NS = 32
DP = 96
NCH = 3
CW = 32
NCORE = 2
NSUB = 16

EB = 2000
EB1 = 2048
NB = 2000

NP8 = 50048
EP = 802816
PER_CORE = EP // NCORE
PER_TILE = PER_CORE // NSUB
BR = 64
NBLK = PER_TILE // BR
NB4 = NBLK // 4
ACC_STRIPE = NP8 // NSUB
PAD_ROW = N

GPER_W = EP // (NCORE * NSUB)
GBLK = GPER_W // 128
GPAIR = GBLK // 2


def _k1_body(ef_ref, esh_ref, w1_ref, b1_ref, w2_ref, b2_ref, wsh_ref, m_ref):
    h = jnp.maximum(ef_ref[...] @ w1_ref[...] + b1_ref[...], 0.0)
    w = h @ w2_ref[...] + b2_ref[...]
    shp = esh_ref[...] @ wsh_ref[...]
    m = w * shp
    lane = lax.broadcasted_iota(jnp.int32, m.shape, 1)
    m_ref[...] = jnp.where(lane == D, 1.0, jnp.where(lane > D, 0.0, m))


def _k1(ef, esh, w1, b1r, w2p, b2p, wshp):
    return pl.pallas_call(
        _k1_body,
        grid=(pl.cdiv(E, EB1),),
        in_specs=[
            pl.BlockSpec((EB1, H), lambda i: (i, 0)),
            pl.BlockSpec((EB1, SH), lambda i: (i, 0)),
            pl.BlockSpec((H, H), lambda i: (0, 0)),
            pl.BlockSpec((1, H), lambda i: (0, 0)),
            pl.BlockSpec((H, DP), lambda i: (0, 0)),
            pl.BlockSpec((1, DP), lambda i: (0, 0)),
            pl.BlockSpec((SH, DP), lambda i: (0, 0)),
        ],
        out_specs=pl.BlockSpec((EB1, DP), lambda i: (i, 0)),
        out_shape=jax.ShapeDtypeStruct((EP, DP), jnp.float32),
    )(ef, esh, w1, b1r, w2p, b2p, wshp)


def _k3a_body(sums_ref, x_ref, out_ref, ssq_ref):
    i = pl.program_id(0)
    sab = sums_ref[...]
    s = sab[0] + sab[1]
    cnt = jnp.maximum(s[:, D:D + 1], 1.0)
    out = s[:, :D] / cnt + x_ref[...]
    out_ref[...] = out

    @pl.when(i == 0)
    def _init():
        ssq_ref[...] = jnp.zeros_like(ssq_ref)

    ssq_ref[...] += jnp.sum(out * out, axis=0, keepdims=True)


def _k3a(sums, x):
    return pl.pallas_call(
        _k3a_body,
        grid=(N // NB,),
        in_specs=[
            pl.BlockSpec((NCORE, NB, DP), lambda i: (0, i, 0)),
            pl.BlockSpec((NB, D), lambda i: (i, 0)),
        ],
        out_specs=[
            pl.BlockSpec((NB, D), lambda i: (i, 0)),
            pl.BlockSpec((1, D), lambda i: (0, 0)),
        ],
        out_shape=[
            jax.ShapeDtypeStruct((N, D), jnp.float32),
            jax.ShapeDtypeStruct((1, D), jnp.float32),
        ],
    )(sums, x)


def _k3b_body(ou_ref, ssq_ref, wlin_ref, out_ref, s_ref):
    scale = lax.rsqrt(ssq_ref[...] / N + 1e-5)
    out = ou_ref[...] * scale
    out_ref[...] = out
    s_ref[...] = out @ wlin_ref[...]


def _k3b(out_un, ssq, wlin):
    return pl.pallas_call(
        _k3b_body,
        grid=(N // NB,),
        in_specs=[
            pl.BlockSpec((NB, D), lambda i: (i, 0)),
            pl.BlockSpec((1, D), lambda i: (0, 0)),
            pl.BlockSpec((D, NS), lambda i: (0, 0)),
        ],
        out_specs=[
            pl.BlockSpec((NB, D), lambda i: (i, 0)),
            pl.BlockSpec((NB, NS), lambda i: (i, 0)),
        ],
        out_shape=[
            jax.ShapeDtypeStruct((N, D), jnp.float32),
            jax.ShapeDtypeStruct((N, NS), jnp.float32),
        ],
    )(out_un, ssq, wlin)


def _k5_body(sd_ref, ss_ref, ef_ref, wa_ref, wb_ref, wc_ref, bu1_ref,
             wu2_ref, bu2_ref, g_ref, b_ref, ef2_ref):
    ef = ef_ref[...]
    pre = (sd_ref[...] @ wa_ref[...] + ss_ref[...] @ wb_ref[...]
           + ef @ wc_ref[...] + bu1_ref[...])
    h1 = jnp.maximum(pre, 0.0)
    h = h1 @ wu2_ref[...] + bu2_ref[...]
    ef2 = ef + h
    mu = jnp.mean(ef2, axis=-1, keepdims=True)
    d = ef2 - mu
    var = jnp.mean(d * d, axis=-1, keepdims=True)
    ef2_ref[...] = g_ref[...] * d * lax.rsqrt(var + 1e-5) + b_ref[...]


def _k5(sd, ss, ef, wa, wb, wc, bu1r, wu2, bu2r, gr, br):
    return pl.pallas_call(
        _k5_body,
        grid=(E // EB,),
        in_specs=[
            pl.BlockSpec((EB, NS), lambda i: (i, 0)),
            pl.BlockSpec((EB, NS), lambda i: (i, 0)),
            pl.BlockSpec((EB, H), lambda i: (i, 0)),
            pl.BlockSpec((NS, H), lambda i: (0, 0)),
            pl.BlockSpec((NS, H), lambda i: (0, 0)),
            pl.BlockSpec((H, H), lambda i: (0, 0)),
            pl.BlockSpec((1, H), lambda i: (0, 0)),
            pl.BlockSpec((H, H), lambda i: (0, 0)),
            pl.BlockSpec((1, H), lambda i: (0, 0)),
            pl.BlockSpec((1, H), lambda i: (0, 0)),
            pl.BlockSpec((1, H), lambda i: (0, 0)),
        ],
        out_specs=pl.BlockSpec((EB, H), lambda i: (i, 0)),
        out_shape=jax.ShapeDtypeStruct((E, H), jnp.float32),
    )(sd, ss, ef, wa, wb, wc, bu1r, wu2, bu2r, gr, br)


def _mulblk(xr, mr, pr):
    for r in range(BR):
        pr[r, pl.ds(0, 16)] = xr[r, pl.ds(0, 16)] * mr[r, pl.ds(0, 16)]
        pr[r, pl.ds(16, 16)] = xr[r, pl.ds(16, 16)] * mr[r, pl.ds(16, 16)]


def _sc_scatter_body(dst_h, src_h, xflat_h, m_h, sums_h, acc,
                     dv0, dv1, dv2, dv3, sv0, sv1, sv2, sv3,
                     xr0, xr1, xr2, xr3, mr0, mr1, mr2, mr3,
                     pr0, pr1, zb,
                     l0, l1, l2, l3, g0, g1, g2, g3, c0, c1):
    core = lax.axis_index("c")
    tec = lax.axis_index("s")
    dvs = (dv0, dv1, dv2, dv3)
    svs = (sv0, sv1, sv2, sv3)
    xrs = (xr0, xr1, xr2, xr3)
    mrs = (mr0, mr1, mr2, mr3)
    prs = (pr0, pr1)
    lsem = (l0, l1, l2, l3)
    gsem = (g0, g1, g2, g3)
    csem = (c0, c1)
    zero16 = jnp.zeros((16,), jnp.float32)
    for r in range(128):
        zb[r, pl.ds(0, 16)] = zero16
        zb[r, pl.ds(16, 16)] = zero16
    base = core * PER_CORE + tec * PER_TILE
    astripe = tec * ACC_STRIPE

    def issue_ld(k, e0, moff):
        pltpu.async_copy(dst_h.at[pl.ds(e0, BR)], dvs[k], lsem[k])
        pltpu.async_copy(src_h.at[pl.ds(e0, BR)], svs[k], lsem[k])
        pltpu.async_copy(m_h.at[pl.ds(e0, BR), pl.ds(moff, CW)],
                         mrs[k], lsem[k])

    def wait_ld(k, e0, moff):
        pltpu.make_async_copy(dst_h.at[pl.ds(e0, BR)], dvs[k], lsem[k]).wait()
        pltpu.make_async_copy(src_h.at[pl.ds(e0, BR)], svs[k], lsem[k]).wait()
        pltpu.make_async_copy(m_h.at[pl.ds(e0, BR), pl.ds(moff, CW)],
                              mrs[k], lsem[k]).wait()

    def issue_g(k, goff):
        for j in range(BR // 16):
            dvs[k][pl.ds(j * 16, 16)] = dvs[k][pl.ds(j * 16, 16)] + goff
        pltpu.async_copy(xflat_h.at[dvs[k]], xrs[k], gsem[k])

    def wait_g(k):
        pltpu.make_async_copy(xflat_h.at[dvs[k]], xrs[k], gsem[k]).wait()

    def issue_sc(k):
        j = k % 2
        _mulblk(xrs[k], mrs[k], prs[j])
        pltpu.async_copy(prs[j], acc.at[svs[k]], csem[j], add=True)

    def wait_sc(j):
        pltpu.make_async_copy(prs[j], acc.at[svs[j]], csem[j]).wait()

    def one_pass(p, _):
        goff = p * NP8
        moff = p * CW
        for k in range(24):
            pltpu.sync_copy(zb, acc.at[pl.ds(astripe + k * 128, 128)])
        pltpu.sync_copy(zb.at[pl.ds(0, 56)],
                        acc.at[pl.ds(astripe + 24 * 128, 56)])
        plsc.subcore_barrier()

        issue_ld(0, base, moff)
        issue_ld(1, base + BR, moff)

        def super_iter(gi, carry):
            e0 = base + gi * (4 * BR)
            wait_ld(0, e0, moff)
            issue_g(0, goff)
            wait_ld(1, e0 + BR, moff)
            issue_g(1, goff)
            issue_ld(2, e0 + 2 * BR, moff)
            issue_ld(3, e0 + 3 * BR, moff)
            wait_g(0)
            issue_sc(0)
            wait_g(1)
            issue_sc(1)
            wait_ld(2, e0 + 2 * BR, moff)
            issue_g(2, goff)
            wait_ld(3, e0 + 3 * BR, moff)
            issue_g(3, goff)
            wait_sc(0)
            wait_g(2)
            issue_sc(2)
            wait_sc(1)
            wait_g(3)
            issue_sc(3)
            wait_sc(0)
            wait_sc(1)

            @pl.when(gi < NB4 - 1)
            def _prefetch():
                issue_ld(0, e0 + 4 * BR, moff)
                issue_ld(1, e0 + 5 * BR, moff)

            return carry

        lax.fori_loop(0, NB4, super_iter, 0)
        plsc.subcore_barrier()
        pltpu.sync_copy(
            acc.at[pl.ds(astripe, ACC_STRIPE)],
            sums_h.at[core, pl.ds(astripe, ACC_STRIPE), pl.ds(moff, CW)])
        plsc.subcore_barrier()
        return _

    lax.fori_loop(0, NCH, one_pass, 0)


def _sc_scatter(dst, src, xflat, m):
    mesh = plsc.VectorSubcoreMesh(core_axis_name="c", subcore_axis_name="s")
    f = pl.kernel(
        _sc_scatter_body,
        out_type=jax.ShapeDtypeStruct((NCORE, NP8, DP), jnp.float32),
        mesh=mesh,
        scratch_types=(
            [pltpu.VMEM_SHARED((NP8, CW), jnp.float32)]
            + [pltpu.VMEM((BR,), jnp.int32)] * 8
            + [pltpu.VMEM((BR, CW), jnp.float32)] * 8
            + [pltpu.VMEM((BR, CW), jnp.float32)] * 2
            + [pltpu.VMEM((128, CW), jnp.float32)]
            + [pltpu.SemaphoreType.DMA] * 10
        ),
        compiler_params=pltpu.CompilerParams(use_tc_tiling_on_sc=False),
    )
    return f(dst, src, xflat, m)


def _sc_gather_body(dst_h, src_h, s_h, sd_h, ss_h,
                    ivd0, ivs0, ivd1, ivs1, rd0, rs0, rd1, rs1,
                    l0, l1, g0, g1, g2, g3, w0, w1, w2, w3):
    core = lax.axis_index("c")
    tec = lax.axis_index("s")
    base = (tec * NCORE + core) * GPER_W

    def issue_ld(k, e0):
        iv_d, iv_s, sem = ((ivd0, ivs0, l0), (ivd1, ivs1, l1))[k]
        pltpu.async_copy(dst_h.at[pl.ds(e0, 128)], iv_d, sem)
        pltpu.async_copy(src_h.at[pl.ds(e0, 128)], iv_s, sem)

    def wait_ld(k, e0):
        iv_d, iv_s, sem = ((ivd0, ivs0, l0), (ivd1, ivs1, l1))[k]
        pltpu.make_async_copy(dst_h.at[pl.ds(e0, 128)], iv_d, sem).wait()
        pltpu.make_async_copy(src_h.at[pl.ds(e0, 128)], iv_s, sem).wait()

    issue_ld(0, base)

    def pair(gi, carry):
        e0 = base + gi * 256
        e1 = e0 + 128
        wait_ld(0, e0)
        pltpu.async_copy(s_h.at[ivd0], rd0, g0)
        pltpu.async_copy(s_h.at[ivs0], rs0, g1)
        issue_ld(1, e1)
        pltpu.make_async_copy(s_h.at[ivd0], rd0, g0).wait()
        pltpu.async_copy(rd0, sd_h.at[pl.ds(e0, 128)], w0)
        pltpu.make_async_copy(s_h.at[ivs0], rs0, g1).wait()
        pltpu.async_copy(rs0, ss_h.at[pl.ds(e0, 128)], w1)
        wait_ld(1, e1)
        pltpu.async_copy(s_h.at[ivd1], rd1, g2)
        pltpu.async_copy(s_h.at[ivs1], rs1, g3)

        @pl.when(gi < GPAIR - 1)
        def _prefetch():
            issue_ld(0, e0 + 256)

        pltpu.make_async_copy(s_h.at[ivd1], rd1, g2).wait()
        pltpu.async_copy(rd1, sd_h.at[pl.ds(e1, 128)], w2)
        pltpu.make_async_copy(s_h.at[ivs1], rs1, g3).wait()
        pltpu.async_copy(rs1, ss_h.at[pl.ds(e1, 128)], w3)
        pltpu.make_async_copy(rd0, sd_h.at[pl.ds(e0, 128)], w0).wait()
        pltpu.make_async_copy(rs0, ss_h.at[pl.ds(e0, 128)], w1).wait()
        pltpu.make_async_copy(rd1, sd_h.at[pl.ds(e1, 128)], w2).wait()
        pltpu.make_async_copy(rs1, ss_h.at[pl.ds(e1, 128)], w3).wait()
        return carry

    lax.fori_loop(0, GPAIR, pair, 0)


def _sc_gather(dst, src, s_pad):
    mesh = plsc.VectorSubcoreMesh(core_axis_name="c", subcore_axis_name="s")
    f = pl.kernel(
        _sc_gather_body,
        out_type=(
            jax.ShapeDtypeStruct((EP, NS), jnp.float32),
            jax.ShapeDtypeStruct((EP, NS), jnp.float32),
        ),
        mesh=mesh,
        scratch_types=(
            [pltpu.VMEM((128,), jnp.int32)] * 4
            + [pltpu.VMEM((128, NS), jnp.float32)] * 4
            + [pltpu.SemaphoreType.DMA] * 10
        ),
        compiler_params=pltpu.CompilerParams(use_tc_tiling_on_sc=False),
    )
    return f(dst, src, s_pad)


def _chunk_table(x):
    xp = jnp.concatenate(
        [x, jnp.ones((N, 1), jnp.float32),
         jnp.zeros((N, DP - D - 1), jnp.float32)], axis=1)
    xp = jnp.concatenate([xp, jnp.zeros((NP8 - N, DP), jnp.float32)], axis=0)
    return xp.reshape(NP8, NCH, CW).transpose(1, 0, 2).reshape(NCH * NP8, CW)


def kernel(atom_features, edge_features, edge_sh, edge_index, W_sh, W1, b1,
           W2, b2, W_lin, Wu1, bu1, Wu2, bu2, gamma, beta):
    dst = edge_index[0]
    src = edge_index[1]
    pad_idx = jnp.full((EP - E,), PAD_ROW, jnp.int32)
    dst_p = jnp.concatenate([dst, pad_idx])
    src_p = jnp.concatenate([src, pad_idx])

    w2p = jnp.pad(W2, ((0, 0), (0, DP - D)))
    b2p = jnp.pad(b2, (0, DP - D)).reshape(1, DP)
    wshp = jnp.pad(W_sh, ((0, 0), (0, DP - D)))
    b1r = b1.reshape(1, H)
    wa, wb, wc = Wu1[:NS], Wu1[NS:2 * NS], Wu1[2 * NS:]
    bu1r = bu1.reshape(1, H)
    bu2r = bu2.reshape(1, H)
    gr = gamma.reshape(1, H)
    br = beta.reshape(1, H)

    x = atom_features
    ef = edge_features
    for layer in range(2):
        m = _k1(ef, edge_sh, W1, b1r, w2p, b2p, wshp)
        xflat = _chunk_table(x)
        sums = _sc_scatter(dst_p, src_p, xflat, m)
        out_un, ssq = _k3a(sums, x)
        out, s = _k3b(out_un, ssq, W_lin)
        x = out
        if layer == 0:
            s_pad = jnp.concatenate(
                [s, jnp.zeros((NP8 - N, NS), jnp.float32)], axis=0)
            sd, ss = _sc_gather(dst_p, src_p, s_pad)
            ef = _k5(sd, ss, ef, wa, wb, wc, bu1r, Wu2, bu2r, gr, br)
    return x


# speedup vs baseline: 2.8380x; 1.1611x over previous
import jax
import jax.numpy as jnp
from jax import lax
from jax.experimental import pallas as pl
from jax.experimental.pallas import tpu as pltpu
from jax.experimental.pallas import tpu_sc as plsc

N = 50000
E = 800000
D = 88
H = 128
SH = 4
NS = 32
DP = 96
NCH = 3
CW = 32
NCORE = 2
NSUB = 16

EB = 2000
EB1 = 2048
NB = 2000

NP8 = 50048
EP = 802816
PER_CORE = EP // NCORE
PER_TILE = PER_CORE // NSUB
BR = 64
NBLK = PER_TILE // BR
NB4 = NBLK // 4
ACC_STRIPE = NP8 // NSUB
PAD_ROW = N

GPER_W = EP // (NCORE * NSUB)
GBLK = GPER_W // 128
GPAIR = GBLK // 2


MW = 128


def _k1_body(ef_ref, esh_ref, w1_ref, b1_ref, w2_ref, b2_ref, wsh_ref, m_ref):
    h = jnp.maximum(ef_ref[...] @ w1_ref[...] + b1_ref[...], 0.0)
    w = h @ w2_ref[...] + b2_ref[...]
    shp = esh_ref[...] @ wsh_ref[...]
    m = w * shp
    lane = lax.broadcasted_iota(jnp.int32, m.shape, 1)
    m_ref[...] = jnp.where(lane == D, 1.0, jnp.where(lane > D, 0.0, m))


def _k1(ef, esh, w1, b1r, w2p, b2p, wshp):
    return pl.pallas_call(
        _k1_body,
        grid=(pl.cdiv(E, EB1),),
        in_specs=[
            pl.BlockSpec((EB1, H), lambda i: (i, 0)),
            pl.BlockSpec((EB1, SH), lambda i: (i, 0)),
            pl.BlockSpec((H, H), lambda i: (0, 0)),
            pl.BlockSpec((1, H), lambda i: (0, 0)),
            pl.BlockSpec((H, MW), lambda i: (0, 0)),
            pl.BlockSpec((1, MW), lambda i: (0, 0)),
            pl.BlockSpec((SH, MW), lambda i: (0, 0)),
        ],
        out_specs=pl.BlockSpec((EB1, MW), lambda i: (i, 0)),
        out_shape=jax.ShapeDtypeStruct((EP, MW), jnp.float32),
    )(ef, esh, w1, b1r, w2p, b2p, wshp)


def _k3a_body(sums_ref, x_ref, out_ref, ssq_ref):
    i = pl.program_id(0)
    sab = sums_ref[...]
    s = sab[0] + sab[1]
    cnt = jnp.maximum(s[:, D:D + 1], 1.0)
    out = s[:, :D] / cnt + x_ref[...]
    out_ref[...] = out

    @pl.when(i == 0)
    def _init():
        ssq_ref[...] = jnp.zeros_like(ssq_ref)

    ssq_ref[...] += jnp.sum(out * out, axis=0, keepdims=True)


def _k3a(sums, x):
    return pl.pallas_call(
        _k3a_body,
        grid=(N // NB,),
        in_specs=[
            pl.BlockSpec((NCORE, NB, DP), lambda i: (0, i, 0)),
            pl.BlockSpec((NB, D), lambda i: (i, 0)),
        ],
        out_specs=[
            pl.BlockSpec((NB, D), lambda i: (i, 0)),
            pl.BlockSpec((1, D), lambda i: (0, 0)),
        ],
        out_shape=[
            jax.ShapeDtypeStruct((N, D), jnp.float32),
            jax.ShapeDtypeStruct((1, D), jnp.float32),
        ],
    )(sums, x)


def _k3b_body(ou_ref, ssq_ref, wlin_ref, out_ref, s_ref):
    scale = lax.rsqrt(ssq_ref[...] / N + 1e-5)
    out = ou_ref[...] * scale
    out_ref[...] = out
    s_ref[...] = out @ wlin_ref[...]


def _k3b(out_un, ssq, wlin):
    return pl.pallas_call(
        _k3b_body,
        grid=(N // NB,),
        in_specs=[
            pl.BlockSpec((NB, D), lambda i: (i, 0)),
            pl.BlockSpec((1, D), lambda i: (0, 0)),
            pl.BlockSpec((D, NS), lambda i: (0, 0)),
        ],
        out_specs=[
            pl.BlockSpec((NB, D), lambda i: (i, 0)),
            pl.BlockSpec((NB, NS), lambda i: (i, 0)),
        ],
        out_shape=[
            jax.ShapeDtypeStruct((N, D), jnp.float32),
            jax.ShapeDtypeStruct((N, NS), jnp.float32),
        ],
    )(out_un, ssq, wlin)


def _k5_body(sd_ref, ss_ref, ef_ref, wa_ref, wb_ref, wc_ref, bu1_ref,
             wu2_ref, bu2_ref, g_ref, b_ref, ef2_ref):
    ef = ef_ref[...]
    pre = (sd_ref[...] @ wa_ref[...] + ss_ref[...] @ wb_ref[...]
           + ef @ wc_ref[...] + bu1_ref[...])
    h1 = jnp.maximum(pre, 0.0)
    h = h1 @ wu2_ref[...] + bu2_ref[...]
    ef2 = ef + h
    mu = jnp.mean(ef2, axis=-1, keepdims=True)
    d = ef2 - mu
    var = jnp.mean(d * d, axis=-1, keepdims=True)
    ef2_ref[...] = g_ref[...] * d * lax.rsqrt(var + 1e-5) + b_ref[...]


def _k5(sd, ss, ef, wa, wb, wc, bu1r, wu2, bu2r, gr, br):
    return pl.pallas_call(
        _k5_body,
        grid=(E // EB,),
        in_specs=[
            pl.BlockSpec((EB, NS), lambda i: (i, 0)),
            pl.BlockSpec((EB, NS), lambda i: (i, 0)),
            pl.BlockSpec((EB, H), lambda i: (i, 0)),
            pl.BlockSpec((NS, H), lambda i: (0, 0)),
            pl.BlockSpec((NS, H), lambda i: (0, 0)),
            pl.BlockSpec((H, H), lambda i: (0, 0)),
            pl.BlockSpec((1, H), lambda i: (0, 0)),
            pl.BlockSpec((H, H), lambda i: (0, 0)),
            pl.BlockSpec((1, H), lambda i: (0, 0)),
            pl.BlockSpec((1, H), lambda i: (0, 0)),
            pl.BlockSpec((1, H), lambda i: (0, 0)),
        ],
        out_specs=pl.BlockSpec((EB, H), lambda i: (i, 0)),
        out_shape=jax.ShapeDtypeStruct((E, H), jnp.float32),
    )(sd, ss, ef, wa, wb, wc, bu1r, wu2, bu2r, gr, br)


def _mulblk(xr, mr, pr):
    for r in range(BR):
        pr[r, pl.ds(0, 16)] = xr[r, pl.ds(0, 16)] * mr[r, pl.ds(0, 16)]
        pr[r, pl.ds(16, 16)] = xr[r, pl.ds(16, 16)] * mr[r, pl.ds(16, 16)]


def _sc_scatter_body(dst_h, src_h, xflat_h, m_h, sums_h, acc,
                     dv0, dv1, dv2, dv3, sv0, sv1, sv2, sv3,
                     xr0, xr1, xr2, xr3, mr0, mr1, mr2, mr3,
                     pr0, pr1, zb,
                     l0, l1, l2, l3, g0, g1, g2, g3, c0, c1):
    core = lax.axis_index("c")
    tec = lax.axis_index("s")
    dvs = (dv0, dv1, dv2, dv3)
    svs = (sv0, sv1, sv2, sv3)
    xrs = (xr0, xr1, xr2, xr3)
    mrs = (mr0, mr1, mr2, mr3)
    prs = (pr0, pr1)
    lsem = (l0, l1, l2, l3)
    gsem = (g0, g1, g2, g3)
    csem = (c0, c1)
    zero16 = jnp.zeros((16,), jnp.float32)
    for r in range(128):
        zb[r, pl.ds(0, 16)] = zero16
        zb[r, pl.ds(16, 16)] = zero16
    base = core * PER_CORE + tec * PER_TILE
    astripe = tec * ACC_STRIPE

    def issue_ld(k, e0, moff):
        pltpu.async_copy(dst_h.at[pl.ds(e0, BR)], dvs[k], lsem[k])
        pltpu.async_copy(src_h.at[pl.ds(e0, BR)], svs[k], lsem[k])
        pltpu.async_copy(m_h.at[pl.ds(e0, BR), pl.ds(moff, CW)],
                         mrs[k], lsem[k])

    def wait_ld(k, e0, moff):
        pltpu.make_async_copy(dst_h.at[pl.ds(e0, BR)], dvs[k], lsem[k]).wait()
        pltpu.make_async_copy(src_h.at[pl.ds(e0, BR)], svs[k], lsem[k]).wait()
        pltpu.make_async_copy(m_h.at[pl.ds(e0, BR), pl.ds(moff, CW)],
                              mrs[k], lsem[k]).wait()

    def issue_g(k, goff):
        for j in range(BR // 16):
            dvs[k][pl.ds(j * 16, 16)] = dvs[k][pl.ds(j * 16, 16)] + goff
        pltpu.async_copy(xflat_h.at[dvs[k]], xrs[k], gsem[k])

    def wait_g(k):
        pltpu.make_async_copy(xflat_h.at[dvs[k]], xrs[k], gsem[k]).wait()

    def issue_sc(k):
        j = k % 2
        _mulblk(xrs[k], mrs[k], prs[j])
        pltpu.async_copy(prs[j], acc.at[svs[k]], csem[j], add=True)

    def wait_sc(j):
        pltpu.make_async_copy(prs[j], acc.at[svs[j]], csem[j]).wait()

    def one_pass(p, _):
        goff = p * NP8
        moff = p * CW
        for k in range(24):
            pltpu.sync_copy(zb, acc.at[pl.ds(astripe + k * 128, 128)])
        pltpu.sync_copy(zb.at[pl.ds(0, 56)],
                        acc.at[pl.ds(astripe + 24 * 128, 56)])
        plsc.subcore_barrier()

        issue_ld(0, base, moff)
        issue_ld(1, base + BR, moff)

        def super_iter(gi, carry):
            e0 = base + gi * (4 * BR)
            wait_ld(0, e0, moff)
            issue_g(0, goff)
            wait_ld(1, e0 + BR, moff)
            issue_g(1, goff)
            issue_ld(2, e0 + 2 * BR, moff)
            issue_ld(3, e0 + 3 * BR, moff)
            wait_g(0)
            issue_sc(0)
            wait_g(1)
            issue_sc(1)
            wait_ld(2, e0 + 2 * BR, moff)
            issue_g(2, goff)
            wait_ld(3, e0 + 3 * BR, moff)
            issue_g(3, goff)
            wait_sc(0)
            wait_g(2)
            issue_sc(2)
            wait_sc(1)
            wait_g(3)
            issue_sc(3)
            wait_sc(0)
            wait_sc(1)

            @pl.when(gi < NB4 - 1)
            def _prefetch():
                issue_ld(0, e0 + 4 * BR, moff)
                issue_ld(1, e0 + 5 * BR, moff)

            return carry

        lax.fori_loop(0, NB4, super_iter, 0)
        plsc.subcore_barrier()
        pltpu.sync_copy(
            acc.at[pl.ds(astripe, ACC_STRIPE)],
            sums_h.at[core, pl.ds(astripe, ACC_STRIPE), pl.ds(moff, CW)])
        plsc.subcore_barrier()
        return _

    lax.fori_loop(0, NCH, one_pass, 0)


def _sc_scatter(dst, src, xflat, m):
    mesh = plsc.VectorSubcoreMesh(core_axis_name="c", subcore_axis_name="s")
    f = pl.kernel(
        _sc_scatter_body,
        out_type=jax.ShapeDtypeStruct((NCORE, NP8, DP), jnp.float32),
        mesh=mesh,
        scratch_types=(
            [pltpu.VMEM_SHARED((NP8, CW), jnp.float32)]
            + [pltpu.VMEM((BR,), jnp.int32)] * 8
            + [pltpu.VMEM((BR, CW), jnp.float32)] * 8
            + [pltpu.VMEM((BR, CW), jnp.float32)] * 2
            + [pltpu.VMEM((128, CW), jnp.float32)]
            + [pltpu.SemaphoreType.DMA] * 10
        ),
        compiler_params=pltpu.CompilerParams(use_tc_tiling_on_sc=False),
    )
    return f(dst, src, xflat, m)


def _sc_gather_body(dst_h, src_h, s_h, sd_h, ss_h,
                    ivd0, ivs0, ivd1, ivs1, rd0, rs0, rd1, rs1,
                    l0, l1, g0, g1, g2, g3, w0, w1, w2, w3):
    core = lax.axis_index("c")
    tec = lax.axis_index("s")
    base = (tec * NCORE + core) * GPER_W

    def issue_ld(k, e0):
        iv_d, iv_s, sem = ((ivd0, ivs0, l0), (ivd1, ivs1, l1))[k]
        pltpu.async_copy(dst_h.at[pl.ds(e0, 128)], iv_d, sem)
        pltpu.async_copy(src_h.at[pl.ds(e0, 128)], iv_s, sem)

    def wait_ld(k, e0):
        iv_d, iv_s, sem = ((ivd0, ivs0, l0), (ivd1, ivs1, l1))[k]
        pltpu.make_async_copy(dst_h.at[pl.ds(e0, 128)], iv_d, sem).wait()
        pltpu.make_async_copy(src_h.at[pl.ds(e0, 128)], iv_s, sem).wait()

    issue_ld(0, base)

    def pair(gi, carry):
        e0 = base + gi * 256
        e1 = e0 + 128
        wait_ld(0, e0)
        pltpu.async_copy(s_h.at[ivd0], rd0, g0)
        pltpu.async_copy(s_h.at[ivs0], rs0, g1)
        issue_ld(1, e1)
        pltpu.make_async_copy(s_h.at[ivd0], rd0, g0).wait()
        pltpu.async_copy(rd0, sd_h.at[pl.ds(e0, 128)], w0)
        pltpu.make_async_copy(s_h.at[ivs0], rs0, g1).wait()
        pltpu.async_copy(rs0, ss_h.at[pl.ds(e0, 128)], w1)
        wait_ld(1, e1)
        pltpu.async_copy(s_h.at[ivd1], rd1, g2)
        pltpu.async_copy(s_h.at[ivs1], rs1, g3)

        @pl.when(gi < GPAIR - 1)
        def _prefetch():
            issue_ld(0, e0 + 256)

        pltpu.make_async_copy(s_h.at[ivd1], rd1, g2).wait()
        pltpu.async_copy(rd1, sd_h.at[pl.ds(e1, 128)], w2)
        pltpu.make_async_copy(s_h.at[ivs1], rs1, g3).wait()
        pltpu.async_copy(rs1, ss_h.at[pl.ds(e1, 128)], w3)
        pltpu.make_async_copy(rd0, sd_h.at[pl.ds(e0, 128)], w0).wait()
        pltpu.make_async_copy(rs0, ss_h.at[pl.ds(e0, 128)], w1).wait()
        pltpu.make_async_copy(rd1, sd_h.at[pl.ds(e1, 128)], w2).wait()
        pltpu.make_async_copy(rs1, ss_h.at[pl.ds(e1, 128)], w3).wait()
        return carry

    lax.fori_loop(0, GPAIR, pair, 0)


def _sc_gather(dst, src, s_pad):
    mesh = plsc.VectorSubcoreMesh(core_axis_name="c", subcore_axis_name="s")
    f = pl.kernel(
        _sc_gather_body,
        out_type=(jax.ShapeDtypeStruct((EP, NS), jnp.float32),
                  jax.ShapeDtypeStruct((EP, NS), jnp.float32)),
        mesh=mesh,
        scratch_types=(
            [pltpu.VMEM((128,), jnp.int32)] * 4
            + [pltpu.VMEM((128, NS), jnp.float32)] * 4
            + [pltpu.SemaphoreType.DMA] * 10
        ),
        compiler_params=pltpu.CompilerParams(use_tc_tiling_on_sc=False),
    )
    return f(dst, src, s_pad)


def _chunk_table(x):
    xp = jnp.concatenate(
        [x, jnp.ones((N, 1), jnp.float32),
         jnp.zeros((N, DP - D - 1), jnp.float32)], axis=1)
    xp = jnp.concatenate([xp, jnp.zeros((NP8 - N, DP), jnp.float32)], axis=0)
    return xp.reshape(NP8, NCH, CW).transpose(1, 0, 2).reshape(NCH * NP8, CW)


def kernel(atom_features, edge_features, edge_sh, edge_index, W_sh, W1, b1,
           W2, b2, W_lin, Wu1, bu1, Wu2, bu2, gamma, beta):
    dst = edge_index[0]
    src = edge_index[1]
    pad_idx = jnp.full((EP - E,), PAD_ROW, jnp.int32)
    dst_p = jnp.concatenate([dst, pad_idx])
    src_p = jnp.concatenate([src, pad_idx])

    w2p = jnp.pad(W2, ((0, 0), (0, MW - D)))
    b2p = jnp.pad(b2, (0, MW - D)).reshape(1, MW)
    wshp = jnp.pad(W_sh, ((0, 0), (0, MW - D)))
    b1r = b1.reshape(1, H)
    wa, wb, wc = Wu1[:NS], Wu1[NS:2 * NS], Wu1[2 * NS:]
    bu1r = bu1.reshape(1, H)
    bu2r = bu2.reshape(1, H)
    gr = gamma.reshape(1, H)
    br = beta.reshape(1, H)

    x = atom_features
    ef = edge_features
    for layer in range(2):
        m = _k1(ef, edge_sh, W1, b1r, w2p, b2p, wshp)
        xflat = _chunk_table(x)
        sums = _sc_scatter(dst_p, src_p, xflat, m)
        out_un, ssq = _k3a(sums, x)
        out, s = _k3b(out_un, ssq, W_lin)
        x = out
        if layer == 0:
            s_pad = jnp.concatenate(
                [s, jnp.zeros((NP8 - N, NS), jnp.float32)], axis=0)
            sd_, ss_ = _sc_gather(dst_p, src_p, s_pad)
            ef = _k5(sd_, ss_, ef, wa, wb, wc, bu1r, Wu2, bu2r, gr, br)
    return x


# speedup vs baseline: 2.9988x; 1.0566x over previous
import jax
import jax.numpy as jnp
from jax import lax
from jax.experimental import pallas as pl
from jax.experimental.pallas import tpu as pltpu
from jax.experimental.pallas import tpu_sc as plsc

N = 50000
E = 800000
D = 88
H = 128
SH = 4
NS = 32
DP = 96
NCH = 3
CW = 32
NCORE = 2
NSUB = 16

EB = 2000
EB1 = 2048
NB = 2000

NP8 = 50048
EP = 802816
PER_CORE = EP // NCORE
PER_TILE = PER_CORE // NSUB
BR = 64
NBLK = PER_TILE // BR
NB4 = NBLK // 4
ACC_STRIPE = NP8 // NSUB
PAD_ROW = N

GPER_W = EP // (NCORE * NSUB)
GBLK = GPER_W // 128
GPAIR = GBLK // 2


MW = 128


def _k1_body(ef_ref, esh_ref, w1_ref, b1_ref, w2_ref, b2_ref, wsh_ref, m_ref):
    h = jnp.maximum(ef_ref[...] @ w1_ref[...] + b1_ref[...], 0.0)
    w = h @ w2_ref[...] + b2_ref[...]
    shp = esh_ref[...] @ wsh_ref[...]
    m = w * shp
    lane = lax.broadcasted_iota(jnp.int32, m.shape, 1)
    m_ref[...] = jnp.where(lane == D, 1.0, jnp.where(lane > D, 0.0, m))


def _k1(ef, esh, w1, b1r, w2p, b2p, wshp):
    return pl.pallas_call(
        _k1_body,
        grid=(pl.cdiv(E, EB1),),
        in_specs=[
            pl.BlockSpec((EB1, H), lambda i: (i, 0)),
            pl.BlockSpec((EB1, SH), lambda i: (i, 0)),
            pl.BlockSpec((H, H), lambda i: (0, 0)),
            pl.BlockSpec((1, H), lambda i: (0, 0)),
            pl.BlockSpec((H, MW), lambda i: (0, 0)),
            pl.BlockSpec((1, MW), lambda i: (0, 0)),
            pl.BlockSpec((SH, MW), lambda i: (0, 0)),
        ],
        out_specs=pl.BlockSpec((EB1, MW), lambda i: (i, 0)),
        out_shape=jax.ShapeDtypeStruct((EP, MW), jnp.float32),
    )(ef, esh, w1, b1r, w2p, b2p, wshp)


def _k3a_body(sums_ref, x_ref, out_ref, ssq_ref):
    i = pl.program_id(0)
    sab = sums_ref[...]
    s = sab[0] + sab[1]
    cnt = jnp.maximum(s[:, D:D + 1], 1.0)
    out = s[:, :D] / cnt + x_ref[...]
    out_ref[...] = out

    @pl.when(i == 0)
    def _init():
        ssq_ref[...] = jnp.zeros_like(ssq_ref)

    ssq_ref[...] += jnp.sum(out * out, axis=0, keepdims=True)


def _k3a(sums, x):
    return pl.pallas_call(
        _k3a_body,
        grid=(N // NB,),
        in_specs=[
            pl.BlockSpec((NCORE, NB, DP), lambda i: (0, i, 0)),
            pl.BlockSpec((NB, D), lambda i: (i, 0)),
        ],
        out_specs=[
            pl.BlockSpec((NB, D), lambda i: (i, 0)),
            pl.BlockSpec((1, D), lambda i: (0, 0)),
        ],
        out_shape=[
            jax.ShapeDtypeStruct((N, D), jnp.float32),
            jax.ShapeDtypeStruct((1, D), jnp.float32),
        ],
    )(sums, x)


def _k3b_body(ou_ref, ssq_ref, wlin_ref, out_ref, s_ref):
    scale = lax.rsqrt(ssq_ref[...] / N + 1e-5)
    out = ou_ref[...] * scale
    out_ref[...] = out
    s_ref[...] = out @ wlin_ref[...]


def _k3b(out_un, ssq, wlin):
    return pl.pallas_call(
        _k3b_body,
        grid=(N // NB,),
        in_specs=[
            pl.BlockSpec((NB, D), lambda i: (i, 0)),
            pl.BlockSpec((1, D), lambda i: (0, 0)),
            pl.BlockSpec((D, NS), lambda i: (0, 0)),
        ],
        out_specs=[
            pl.BlockSpec((NB, D), lambda i: (i, 0)),
            pl.BlockSpec((NB, NS), lambda i: (i, 0)),
        ],
        out_shape=[
            jax.ShapeDtypeStruct((N, D), jnp.float32),
            jax.ShapeDtypeStruct((N, NS), jnp.float32),
        ],
    )(out_un, ssq, wlin)


def _k5_body(sds_ref, ef_ref, wab_ref, wc_ref, bu1_ref,
             wu2_ref, bu2_ref, g_ref, b_ref, ef2_ref):
    ef = ef_ref[...]
    pre = sds_ref[...] @ wab_ref[...] + ef @ wc_ref[...] + bu1_ref[...]
    h1 = jnp.maximum(pre, 0.0)
    h = h1 @ wu2_ref[...] + bu2_ref[...]
    ef2 = ef + h
    mu = jnp.mean(ef2, axis=-1, keepdims=True)
    d = ef2 - mu
    var = jnp.mean(d * d, axis=-1, keepdims=True)
    ef2_ref[...] = g_ref[...] * d * lax.rsqrt(var + 1e-5) + b_ref[...]


def _k5(sds, ef, wab, wc, bu1r, wu2, bu2r, gr, br):
    return pl.pallas_call(
        _k5_body,
        grid=(E // EB,),
        in_specs=[
            pl.BlockSpec((EB, 2 * NS), lambda i: (i, 0)),
            pl.BlockSpec((EB, H), lambda i: (i, 0)),
            pl.BlockSpec((2 * NS, H), lambda i: (0, 0)),
            pl.BlockSpec((H, H), lambda i: (0, 0)),
            pl.BlockSpec((1, H), lambda i: (0, 0)),
            pl.BlockSpec((H, H), lambda i: (0, 0)),
            pl.BlockSpec((1, H), lambda i: (0, 0)),
            pl.BlockSpec((1, H), lambda i: (0, 0)),
            pl.BlockSpec((1, H), lambda i: (0, 0)),
        ],
        out_specs=pl.BlockSpec((EB, H), lambda i: (i, 0)),
        out_shape=jax.ShapeDtypeStruct((E, H), jnp.float32),
    )(sds, ef, wab, wc, bu1r, wu2, bu2r, gr, br)


def _mulblk(xr, mr, pr):
    for r in range(BR):
        pr[r, pl.ds(0, 16)] = xr[r, pl.ds(0, 16)] * mr[r, pl.ds(0, 16)]
        pr[r, pl.ds(16, 16)] = xr[r, pl.ds(16, 16)] * mr[r, pl.ds(16, 16)]


def _sc_scatter_body(dst_h, src_h, xflat_h, m_h, sums_h, acc,
                     dv0, dv1, dv2, dv3, sv0, sv1, sv2, sv3,
                     xr0, xr1, xr2, xr3, mr0, mr1, mr2, mr3,
                     pr0, pr1, zb,
                     l0, l1, l2, l3, g0, g1, g2, g3, c0, c1):
    core = lax.axis_index("c")
    tec = lax.axis_index("s")
    dvs = (dv0, dv1, dv2, dv3)
    svs = (sv0, sv1, sv2, sv3)
    xrs = (xr0, xr1, xr2, xr3)
    mrs = (mr0, mr1, mr2, mr3)
    prs = (pr0, pr1)
    lsem = (l0, l1, l2, l3)
    gsem = (g0, g1, g2, g3)
    csem = (c0, c1)
    zero16 = jnp.zeros((16,), jnp.float32)
    for r in range(128):
        zb[r, pl.ds(0, 16)] = zero16
        zb[r, pl.ds(16, 16)] = zero16
    base = core * PER_CORE + tec * PER_TILE
    astripe = tec * ACC_STRIPE

    def issue_ld(k, e0, moff):
        pltpu.async_copy(dst_h.at[pl.ds(e0, BR)], dvs[k], lsem[k])
        pltpu.async_copy(src_h.at[pl.ds(e0, BR)], svs[k], lsem[k])
        pltpu.async_copy(m_h.at[pl.ds(e0, BR), pl.ds(moff, CW)],
                         mrs[k], lsem[k])

    def wait_ld(k, e0, moff):
        pltpu.make_async_copy(dst_h.at[pl.ds(e0, BR)], dvs[k], lsem[k]).wait()
        pltpu.make_async_copy(src_h.at[pl.ds(e0, BR)], svs[k], lsem[k]).wait()
        pltpu.make_async_copy(m_h.at[pl.ds(e0, BR), pl.ds(moff, CW)],
                              mrs[k], lsem[k]).wait()

    def issue_g(k, goff):
        for j in range(BR // 16):
            dvs[k][pl.ds(j * 16, 16)] = dvs[k][pl.ds(j * 16, 16)] + goff
        pltpu.async_copy(xflat_h.at[dvs[k]], xrs[k], gsem[k])

    def wait_g(k):
        pltpu.make_async_copy(xflat_h.at[dvs[k]], xrs[k], gsem[k]).wait()

    def issue_sc(k):
        j = k % 2
        _mulblk(xrs[k], mrs[k], prs[j])
        pltpu.async_copy(prs[j], acc.at[svs[k]], csem[j], add=True)

    def wait_sc(j):
        pltpu.make_async_copy(prs[j], acc.at[svs[j]], csem[j]).wait()

    def one_pass(p, _):
        goff = p * NP8
        moff = p * CW
        for k in range(24):
            pltpu.sync_copy(zb, acc.at[pl.ds(astripe + k * 128, 128)])
        pltpu.sync_copy(zb.at[pl.ds(0, 56)],
                        acc.at[pl.ds(astripe + 24 * 128, 56)])
        plsc.subcore_barrier()

        issue_ld(0, base, moff)
        issue_ld(1, base + BR, moff)

        def super_iter(gi, carry):
            e0 = base + gi * (4 * BR)
            wait_ld(0, e0, moff)
            issue_g(0, goff)
            wait_ld(1, e0 + BR, moff)
            issue_g(1, goff)
            issue_ld(2, e0 + 2 * BR, moff)
            issue_ld(3, e0 + 3 * BR, moff)
            wait_g(0)
            issue_sc(0)
            wait_g(1)
            issue_sc(1)
            wait_ld(2, e0 + 2 * BR, moff)
            issue_g(2, goff)
            wait_ld(3, e0 + 3 * BR, moff)
            issue_g(3, goff)
            wait_sc(0)
            wait_g(2)
            issue_sc(2)
            wait_sc(1)
            wait_g(3)
            issue_sc(3)
            wait_sc(0)
            wait_sc(1)

            @pl.when(gi < NB4 - 1)
            def _prefetch():
                issue_ld(0, e0 + 4 * BR, moff)
                issue_ld(1, e0 + 5 * BR, moff)

            return carry

        lax.fori_loop(0, NB4, super_iter, 0)
        plsc.subcore_barrier()
        pltpu.sync_copy(
            acc.at[pl.ds(astripe, ACC_STRIPE)],
            sums_h.at[core, pl.ds(astripe, ACC_STRIPE), pl.ds(moff, CW)])
        plsc.subcore_barrier()
        return _

    lax.fori_loop(0, NCH, one_pass, 0)


def _sc_scatter(dst, src, xflat, m):
    mesh = plsc.VectorSubcoreMesh(core_axis_name="c", subcore_axis_name="s")
    f = pl.kernel(
        _sc_scatter_body,
        out_type=jax.ShapeDtypeStruct((NCORE, NP8, DP), jnp.float32),
        mesh=mesh,
        scratch_types=(
            [pltpu.VMEM_SHARED((NP8, CW), jnp.float32)]
            + [pltpu.VMEM((BR,), jnp.int32)] * 8
            + [pltpu.VMEM((BR, CW), jnp.float32)] * 8
            + [pltpu.VMEM((BR, CW), jnp.float32)] * 2
            + [pltpu.VMEM((128, CW), jnp.float32)]
            + [pltpu.SemaphoreType.DMA] * 10
        ),
        compiler_params=pltpu.CompilerParams(use_tc_tiling_on_sc=False),
    )
    return f(dst, src, xflat, m)


def _sc_gather_body(dst_h, src_h, s_h, sds_h,
                    ivd0, ivs0, ivd1, ivs1, rd0, rs0, rd1, rs1,
                    l0, l1, g0, g1, g2, g3, w0, w1, w2, w3):
    core = lax.axis_index("c")
    tec = lax.axis_index("s")
    base = (tec * NCORE + core) * GPER_W

    def issue_ld(k, e0):
        iv_d, iv_s, sem = ((ivd0, ivs0, l0), (ivd1, ivs1, l1))[k]
        pltpu.async_copy(dst_h.at[pl.ds(e0, 128)], iv_d, sem)
        pltpu.async_copy(src_h.at[pl.ds(e0, 128)], iv_s, sem)

    def wait_ld(k, e0):
        iv_d, iv_s, sem = ((ivd0, ivs0, l0), (ivd1, ivs1, l1))[k]
        pltpu.make_async_copy(dst_h.at[pl.ds(e0, 128)], iv_d, sem).wait()
        pltpu.make_async_copy(src_h.at[pl.ds(e0, 128)], iv_s, sem).wait()

    issue_ld(0, base)

    def pair(gi, carry):
        e0 = base + gi * 256
        e1 = e0 + 128
        wait_ld(0, e0)
        pltpu.async_copy(s_h.at[ivd0], rd0, g0)
        pltpu.async_copy(s_h.at[ivs0], rs0, g1)
        issue_ld(1, e1)
        pltpu.make_async_copy(s_h.at[ivd0], rd0, g0).wait()
        pltpu.async_copy(rd0, sds_h.at[pl.ds(e0, 128), pl.ds(0, NS)], w0)
        pltpu.make_async_copy(s_h.at[ivs0], rs0, g1).wait()
        pltpu.async_copy(rs0, sds_h.at[pl.ds(e0, 128), pl.ds(NS, NS)], w1)
        wait_ld(1, e1)
        pltpu.async_copy(s_h.at[ivd1], rd1, g2)
        pltpu.async_copy(s_h.at[ivs1], rs1, g3)

        @pl.when(gi < GPAIR - 1)
        def _prefetch():
            issue_ld(0, e0 + 256)

        pltpu.make_async_copy(s_h.at[ivd1], rd1, g2).wait()
        pltpu.async_copy(rd1, sds_h.at[pl.ds(e1, 128), pl.ds(0, NS)], w2)
        pltpu.make_async_copy(s_h.at[ivs1], rs1, g3).wait()
        pltpu.async_copy(rs1, sds_h.at[pl.ds(e1, 128), pl.ds(NS, NS)], w3)
        pltpu.make_async_copy(rd0, sds_h.at[pl.ds(e0, 128), pl.ds(0, NS)], w0).wait()
        pltpu.make_async_copy(rs0, sds_h.at[pl.ds(e0, 128), pl.ds(NS, NS)], w1).wait()
        pltpu.make_async_copy(rd1, sds_h.at[pl.ds(e1, 128), pl.ds(0, NS)], w2).wait()
        pltpu.make_async_copy(rs1, sds_h.at[pl.ds(e1, 128), pl.ds(NS, NS)], w3).wait()
        return carry

    lax.fori_loop(0, GPAIR, pair, 0)


def _sc_gather(dst, src, s_pad):
    mesh = plsc.VectorSubcoreMesh(core_axis_name="c", subcore_axis_name="s")
    f = pl.kernel(
        _sc_gather_body,
        out_type=jax.ShapeDtypeStruct((EP, 2 * NS), jnp.float32),
        mesh=mesh,
        scratch_types=(
            [pltpu.VMEM((128,), jnp.int32)] * 4
            + [pltpu.VMEM((128, NS), jnp.float32)] * 4
            + [pltpu.SemaphoreType.DMA] * 10
        ),
        compiler_params=pltpu.CompilerParams(use_tc_tiling_on_sc=False),
    )
    return f(dst, src, s_pad)


def _chunk_table(x):
    xp = jnp.concatenate(
        [x, jnp.ones((N, 1), jnp.float32),
         jnp.zeros((N, DP - D - 1), jnp.float32)], axis=1)
    xp = jnp.concatenate([xp, jnp.zeros((NP8 - N, DP), jnp.float32)], axis=0)
    return xp.reshape(NP8, NCH, CW).transpose(1, 0, 2).reshape(NCH * NP8, CW)


def kernel(atom_features, edge_features, edge_sh, edge_index, W_sh, W1, b1,
           W2, b2, W_lin, Wu1, bu1, Wu2, bu2, gamma, beta):
    dst = edge_index[0]
    src = edge_index[1]
    pad_idx = jnp.full((EP - E,), PAD_ROW, jnp.int32)
    dst_p = jnp.concatenate([dst, pad_idx])
    src_p = jnp.concatenate([src, pad_idx])

    w2p = jnp.pad(W2, ((0, 0), (0, MW - D)))
    b2p = jnp.pad(b2, (0, MW - D)).reshape(1, MW)
    wshp = jnp.pad(W_sh, ((0, 0), (0, MW - D)))
    b1r = b1.reshape(1, H)
    wab, wc = Wu1[:2 * NS], Wu1[2 * NS:]
    bu1r = bu1.reshape(1, H)
    bu2r = bu2.reshape(1, H)
    gr = gamma.reshape(1, H)
    br = beta.reshape(1, H)

    x = atom_features
    ef = edge_features
    for layer in range(2):
        m = _k1(ef, edge_sh, W1, b1r, w2p, b2p, wshp)
        xflat = _chunk_table(x)
        sums = _sc_scatter(dst_p, src_p, xflat, m)
        out_un, ssq = _k3a(sums, x)
        out, s = _k3b(out_un, ssq, W_lin)
        x = out
        if layer == 0:
            s_pad = jnp.concatenate(
                [s, jnp.zeros((NP8 - N, NS), jnp.float32)], axis=0)
            sds = _sc_gather(dst_p, src_p, s_pad)
            ef = _k5(sds, ef, wab, wc, bu1r, Wu2, bu2r, gr, br)
    return x


# speedup vs baseline: 3.1875x; 1.0629x over previous
import jax
import jax.numpy as jnp
from jax import lax
from jax.experimental import pallas as pl
from jax.experimental.pallas import tpu as pltpu
from jax.experimental.pallas import tpu_sc as plsc

N = 50000
E = 800000
D = 88
H = 128
SH = 4
NS = 32
DP = 96
NCH = 3
CW = 32
NCORE = 2
NSUB = 16

EB = 2000
EB1 = 2048
NB = 2000

NP8 = 50048
EP = 802816
PER_CORE = EP // NCORE
PER_TILE = PER_CORE // NSUB
BR = 64
NBLK = PER_TILE // BR
NB4 = NBLK // 4
ACC_STRIPE = NP8 // NSUB
PAD_ROW = N

SW = 128
GPER_W = EP // (NCORE * NSUB)
GBLK = GPER_W // 128
GPAIR = GBLK // 2


MW = 128


def _dotf32(a, b):
    return lax.dot_general(a, b, (((1,), (0,)), ((), ())),
                           preferred_element_type=jnp.float32)


def _k1_body(ef_ref, esh_ref, w1_ref, b1_ref, w2_ref, b2_ref, wsh_ref, m_ref):
    ef16 = ef_ref[...].astype(jnp.bfloat16)
    h = jnp.maximum(_dotf32(ef16, w1_ref[...]) + b1_ref[...], 0.0)
    w = _dotf32(h.astype(jnp.bfloat16), w2_ref[...]) + b2_ref[...]
    shp = esh_ref[...] @ wsh_ref[...]
    m = w * shp
    lane = lax.broadcasted_iota(jnp.int32, m.shape, 1)
    m_ref[...] = jnp.where(lane == D, 1.0, jnp.where(lane > D, 0.0, m))


def _k1(ef, esh, w1, b1r, w2p, b2p, wshp):
    return pl.pallas_call(
        _k1_body,
        grid=(pl.cdiv(E, EB1),),
        in_specs=[
            pl.BlockSpec((EB1, H), lambda i: (i, 0)),
            pl.BlockSpec((EB1, SH), lambda i: (i, 0)),
            pl.BlockSpec((H, H), lambda i: (0, 0)),
            pl.BlockSpec((1, H), lambda i: (0, 0)),
            pl.BlockSpec((H, MW), lambda i: (0, 0)),
            pl.BlockSpec((1, MW), lambda i: (0, 0)),
            pl.BlockSpec((SH, MW), lambda i: (0, 0)),
        ],
        out_specs=pl.BlockSpec((EB1, MW), lambda i: (i, 0)),
        out_shape=jax.ShapeDtypeStruct((EP, MW), jnp.float32),
    )(ef, esh, w1, b1r, w2p, b2p, wshp)


def _k3a_body(sums_ref, x_ref, out_ref, ssq_ref):
    i = pl.program_id(0)
    sab = sums_ref[...]
    s = sab[0] + sab[1]
    cnt = jnp.maximum(s[:, D:D + 1], 1.0)
    out = s[:, :D] / cnt + x_ref[...]
    out_ref[...] = out

    @pl.when(i == 0)
    def _init():
        ssq_ref[...] = jnp.zeros_like(ssq_ref)

    ssq_ref[...] += jnp.sum(out * out, axis=0, keepdims=True)


def _k3a(sums, x):
    return pl.pallas_call(
        _k3a_body,
        grid=(N // NB,),
        in_specs=[
            pl.BlockSpec((NCORE, NB, DP), lambda i: (0, i, 0)),
            pl.BlockSpec((NB, D), lambda i: (i, 0)),
        ],
        out_specs=[
            pl.BlockSpec((NB, D), lambda i: (i, 0)),
            pl.BlockSpec((1, D), lambda i: (0, 0)),
        ],
        out_shape=[
            jax.ShapeDtypeStruct((N, D), jnp.float32),
            jax.ShapeDtypeStruct((1, D), jnp.float32),
        ],
    )(sums, x)


def _k3b_body(ou_ref, ssq_ref, wlin_ref, out_ref, s_ref):
    scale = lax.rsqrt(ssq_ref[...] / N + 1e-5)
    out = ou_ref[...] * scale
    out_ref[...] = out
    s_ref[...] = out @ wlin_ref[...]


def _k3b(out_un, ssq, wlin):
    return pl.pallas_call(
        _k3b_body,
        grid=(N // NB,),
        in_specs=[
            pl.BlockSpec((NB, D), lambda i: (i, 0)),
            pl.BlockSpec((1, D), lambda i: (0, 0)),
            pl.BlockSpec((D, NS), lambda i: (0, 0)),
        ],
        out_specs=[
            pl.BlockSpec((NB, D), lambda i: (i, 0)),
            pl.BlockSpec((NB, NS), lambda i: (i, 0)),
        ],
        out_shape=[
            jax.ShapeDtypeStruct((N, D), jnp.float32),
            jax.ShapeDtypeStruct((N, NS), jnp.float32),
        ],
    )(out_un, ssq, wlin)


def _k5_body(sds_ref, ef_ref, wab_ref, wc_ref, bu1_ref,
             wu2_ref, bu2_ref, g_ref, b_ref, ef2_ref):
    ef = ef_ref[...]
    sds16 = sds_ref[..., :2 * NS].astype(jnp.bfloat16)
    pre = (_dotf32(sds16, wab_ref[...]) + _dotf32(ef.astype(jnp.bfloat16), wc_ref[...])
           + bu1_ref[...])
    h1 = jnp.maximum(pre, 0.0)
    h = _dotf32(h1.astype(jnp.bfloat16), wu2_ref[...]) + bu2_ref[...]
    ef2 = ef.astype(jnp.float32) + h
    mu = jnp.mean(ef2, axis=-1, keepdims=True)
    d = ef2 - mu
    var = jnp.mean(d * d, axis=-1, keepdims=True)
    ef2 = g_ref[...] * d * lax.rsqrt(var + 1e-5) + b_ref[...]
    ef2_ref[...] = ef2.astype(jnp.bfloat16)


def _k5(sds, ef, wab, wc, bu1r, wu2, bu2r, gr, br):
    return pl.pallas_call(
        _k5_body,
        grid=(E // EB,),
        in_specs=[
            pl.BlockSpec((EB, SW), lambda i: (i, 0)),
            pl.BlockSpec((EB, H), lambda i: (i, 0)),
            pl.BlockSpec((2 * NS, H), lambda i: (0, 0)),
            pl.BlockSpec((H, H), lambda i: (0, 0)),
            pl.BlockSpec((1, H), lambda i: (0, 0)),
            pl.BlockSpec((H, H), lambda i: (0, 0)),
            pl.BlockSpec((1, H), lambda i: (0, 0)),
            pl.BlockSpec((1, H), lambda i: (0, 0)),
            pl.BlockSpec((1, H), lambda i: (0, 0)),
        ],
        out_specs=pl.BlockSpec((EB, H), lambda i: (i, 0)),
        out_shape=jax.ShapeDtypeStruct((E, H), jnp.bfloat16),
    )(sds, ef, wab, wc, bu1r, wu2, bu2r, gr, br)


def _mulblk(xr, mr, pr):
    for r in range(BR):
        pr[r, pl.ds(0, 16)] = xr[r, pl.ds(0, 16)] * mr[r, pl.ds(0, 16)]
        pr[r, pl.ds(16, 16)] = xr[r, pl.ds(16, 16)] * mr[r, pl.ds(16, 16)]


def _sc_scatter_body(dst_h, src_h, xflat_h, m_h, sums_h, acc,
                     dv0, dv1, dv2, dv3, sv0, sv1, sv2, sv3,
                     xr0, xr1, xr2, xr3, mr0, mr1, mr2, mr3,
                     pr0, pr1, zb,
                     l0, l1, l2, l3, g0, g1, g2, g3, c0, c1):
    core = lax.axis_index("c")
    tec = lax.axis_index("s")
    dvs = (dv0, dv1, dv2, dv3)
    svs = (sv0, sv1, sv2, sv3)
    xrs = (xr0, xr1, xr2, xr3)
    mrs = (mr0, mr1, mr2, mr3)
    prs = (pr0, pr1)
    lsem = (l0, l1, l2, l3)
    gsem = (g0, g1, g2, g3)
    csem = (c0, c1)
    zero16 = jnp.zeros((16,), jnp.float32)
    for r in range(128):
        zb[r, pl.ds(0, 16)] = zero16
        zb[r, pl.ds(16, 16)] = zero16
    base = core * PER_CORE + tec * PER_TILE
    astripe = tec * ACC_STRIPE

    def issue_ld(k, e0, moff):
        pltpu.async_copy(dst_h.at[pl.ds(e0, BR)], dvs[k], lsem[k])
        pltpu.async_copy(src_h.at[pl.ds(e0, BR)], svs[k], lsem[k])
        pltpu.async_copy(m_h.at[pl.ds(e0, BR), pl.ds(moff, CW)],
                         mrs[k], lsem[k])

    def wait_ld(k, e0, moff):
        pltpu.make_async_copy(dst_h.at[pl.ds(e0, BR)], dvs[k], lsem[k]).wait()
        pltpu.make_async_copy(src_h.at[pl.ds(e0, BR)], svs[k], lsem[k]).wait()
        pltpu.make_async_copy(m_h.at[pl.ds(e0, BR), pl.ds(moff, CW)],
                              mrs[k], lsem[k]).wait()

    def issue_g(k, goff):
        for j in range(BR // 16):
            dvs[k][pl.ds(j * 16, 16)] = dvs[k][pl.ds(j * 16, 16)] + goff
        pltpu.async_copy(xflat_h.at[dvs[k]], xrs[k], gsem[k])

    def wait_g(k):
        pltpu.make_async_copy(xflat_h.at[dvs[k]], xrs[k], gsem[k]).wait()

    def issue_sc(k):
        j = k % 2
        _mulblk(xrs[k], mrs[k], prs[j])
        pltpu.async_copy(prs[j], acc.at[svs[k]], csem[j], add=True)

    def wait_sc(j):
        pltpu.make_async_copy(prs[j], acc.at[svs[j]], csem[j]).wait()

    def one_pass(p, _):
        goff = p * NP8
        moff = p * CW
        for k in range(24):
            pltpu.sync_copy(zb, acc.at[pl.ds(astripe + k * 128, 128)])
        pltpu.sync_copy(zb.at[pl.ds(0, 56)],
                        acc.at[pl.ds(astripe + 24 * 128, 56)])
        plsc.subcore_barrier()

        issue_ld(0, base, moff)
        issue_ld(1, base + BR, moff)

        def super_iter(gi, carry):
            e0 = base + gi * (4 * BR)
            wait_ld(0, e0, moff)
            issue_g(0, goff)
            wait_ld(1, e0 + BR, moff)
            issue_g(1, goff)
            issue_ld(2, e0 + 2 * BR, moff)
            issue_ld(3, e0 + 3 * BR, moff)
            wait_g(0)
            issue_sc(0)
            wait_g(1)
            issue_sc(1)
            wait_ld(2, e0 + 2 * BR, moff)
            issue_g(2, goff)
            wait_ld(3, e0 + 3 * BR, moff)
            issue_g(3, goff)
            wait_sc(0)
            wait_g(2)
            issue_sc(2)
            wait_sc(1)
            wait_g(3)
            issue_sc(3)
            wait_sc(0)
            wait_sc(1)

            @pl.when(gi < NB4 - 1)
            def _prefetch():
                issue_ld(0, e0 + 4 * BR, moff)
                issue_ld(1, e0 + 5 * BR, moff)

            return carry

        lax.fori_loop(0, NB4, super_iter, 0)
        plsc.subcore_barrier()
        pltpu.sync_copy(
            acc.at[pl.ds(astripe, ACC_STRIPE)],
            sums_h.at[core, pl.ds(astripe, ACC_STRIPE), pl.ds(moff, CW)])
        plsc.subcore_barrier()
        return _

    lax.fori_loop(0, NCH, one_pass, 0)


def _sc_scatter(dst, src, xflat, m):
    mesh = plsc.VectorSubcoreMesh(core_axis_name="c", subcore_axis_name="s")
    f = pl.kernel(
        _sc_scatter_body,
        out_type=jax.ShapeDtypeStruct((NCORE, NP8, DP), jnp.float32),
        mesh=mesh,
        scratch_types=(
            [pltpu.VMEM_SHARED((NP8, CW), jnp.float32)]
            + [pltpu.VMEM((BR,), jnp.int32)] * 8
            + [pltpu.VMEM((BR, CW), jnp.float32)] * 8
            + [pltpu.VMEM((BR, CW), jnp.float32)] * 2
            + [pltpu.VMEM((128, CW), jnp.float32)]
            + [pltpu.SemaphoreType.DMA] * 10
        ),
        compiler_params=pltpu.CompilerParams(use_tc_tiling_on_sc=False),
    )
    return f(dst, src, xflat, m)


def _sc_gather_body(dst_h, src_h, s_h, sds_h,
                    ivd0, ivs0, ivd1, ivs1, rd0, rs0, rd1, rs1,
                    l0, l1, g0, g1, g2, g3, w0, w1, w2, w3):
    core = lax.axis_index("c")
    tec = lax.axis_index("s")
    base = (tec * NCORE + core) * GPER_W

    def issue_ld(k, e0):
        iv_d, iv_s, sem = ((ivd0, ivs0, l0), (ivd1, ivs1, l1))[k]
        pltpu.async_copy(dst_h.at[pl.ds(e0, 128)], iv_d, sem)
        pltpu.async_copy(src_h.at[pl.ds(e0, 128)], iv_s, sem)

    def wait_ld(k, e0):
        iv_d, iv_s, sem = ((ivd0, ivs0, l0), (ivd1, ivs1, l1))[k]
        pltpu.make_async_copy(dst_h.at[pl.ds(e0, 128)], iv_d, sem).wait()
        pltpu.make_async_copy(src_h.at[pl.ds(e0, 128)], iv_s, sem).wait()

    issue_ld(0, base)

    def pair(gi, carry):
        e0 = base + gi * 256
        e1 = e0 + 128
        wait_ld(0, e0)
        pltpu.async_copy(s_h.at[ivd0], rd0, g0)
        pltpu.async_copy(s_h.at[ivs0], rs0, g1)
        issue_ld(1, e1)
        pltpu.make_async_copy(s_h.at[ivd0], rd0, g0).wait()
        pltpu.async_copy(rd0, sds_h.at[pl.ds(e0, 128), pl.ds(0, NS)], w0)
        pltpu.make_async_copy(s_h.at[ivs0], rs0, g1).wait()
        pltpu.async_copy(rs0, sds_h.at[pl.ds(e0, 128), pl.ds(NS, NS)], w1)
        wait_ld(1, e1)
        pltpu.async_copy(s_h.at[ivd1], rd1, g2)
        pltpu.async_copy(s_h.at[ivs1], rs1, g3)

        @pl.when(gi < GPAIR - 1)
        def _prefetch():
            issue_ld(0, e0 + 256)

        pltpu.make_async_copy(s_h.at[ivd1], rd1, g2).wait()
        pltpu.async_copy(rd1, sds_h.at[pl.ds(e1, 128), pl.ds(0, NS)], w2)
        pltpu.make_async_copy(s_h.at[ivs1], rs1, g3).wait()
        pltpu.async_copy(rs1, sds_h.at[pl.ds(e1, 128), pl.ds(NS, NS)], w3)
        pltpu.make_async_copy(rd0, sds_h.at[pl.ds(e0, 128), pl.ds(0, NS)], w0).wait()
        pltpu.make_async_copy(rs0, sds_h.at[pl.ds(e0, 128), pl.ds(NS, NS)], w1).wait()
        pltpu.make_async_copy(rd1, sds_h.at[pl.ds(e1, 128), pl.ds(0, NS)], w2).wait()
        pltpu.make_async_copy(rs1, sds_h.at[pl.ds(e1, 128), pl.ds(NS, NS)], w3).wait()
        return carry

    lax.fori_loop(0, GPAIR, pair, 0)


def _sc_gather(dst, src, s_pad):
    mesh = plsc.VectorSubcoreMesh(core_axis_name="c", subcore_axis_name="s")
    f = pl.kernel(
        _sc_gather_body,
        out_type=jax.ShapeDtypeStruct((EP, SW), jnp.float32),
        mesh=mesh,
        scratch_types=(
            [pltpu.VMEM((128,), jnp.int32)] * 4
            + [pltpu.VMEM((128, NS), jnp.float32)] * 4
            + [pltpu.SemaphoreType.DMA] * 10
        ),
        compiler_params=pltpu.CompilerParams(use_tc_tiling_on_sc=False),
    )
    return f(dst, src, s_pad)


def _chunk_table(x):
    xp = jnp.concatenate(
        [x, jnp.ones((N, 1), jnp.float32),
         jnp.zeros((N, DP - D - 1), jnp.float32)], axis=1)
    xp = jnp.concatenate([xp, jnp.zeros((NP8 - N, DP), jnp.float32)], axis=0)
    return xp.reshape(NP8, NCH, CW).transpose(1, 0, 2).reshape(NCH * NP8, CW)


def kernel(atom_features, edge_features, edge_sh, edge_index, W_sh, W1, b1,
           W2, b2, W_lin, Wu1, bu1, Wu2, bu2, gamma, beta):
    dst = edge_index[0]
    src = edge_index[1]
    pad_idx = jnp.full((EP - E,), PAD_ROW, jnp.int32)
    dst_p = jnp.concatenate([dst, pad_idx])
    src_p = jnp.concatenate([src, pad_idx])

    w2p = jnp.pad(W2, ((0, 0), (0, MW - D))).astype(jnp.bfloat16)
    b2p = jnp.pad(b2, (0, MW - D)).reshape(1, MW)
    wshp = jnp.pad(W_sh, ((0, 0), (0, MW - D)))
    b1r = b1.reshape(1, H)
    w1h = W1.astype(jnp.bfloat16)
    wab = Wu1[:2 * NS].astype(jnp.bfloat16)
    wc = Wu1[2 * NS:].astype(jnp.bfloat16)
    wu2h = Wu2.astype(jnp.bfloat16)
    bu1r = bu1.reshape(1, H)
    bu2r = bu2.reshape(1, H)
    gr = gamma.reshape(1, H)
    br = beta.reshape(1, H)

    x = atom_features
    ef = edge_features
    for layer in range(2):
        m = _k1(ef, edge_sh, w1h, b1r, w2p, b2p, wshp)
        xflat = _chunk_table(x)
        sums = _sc_scatter(dst_p, src_p, xflat, m)
        out_un, ssq = _k3a(sums, x)
        out, s = _k3b(out_un, ssq, W_lin)
        x = out
        if layer == 0:
            s_pad = jnp.concatenate(
                [s, jnp.zeros((NP8 - N, NS), jnp.float32)], axis=0)
            sds = _sc_gather(dst_p, src_p, s_pad)
            ef = _k5(sds, ef, wab, wc, bu1r, wu2h, bu2r, gr, br)
    return x
